# baseline (speedup 1.0000x reference)
import functools

import jax
import jax.numpy as jnp
from jax import lax
from jax.experimental import pallas as pl
from jax.experimental.pallas import tpu as pltpu
from jax.experimental.pallas import tpu_sc as plsc

_NC = 2
_NS = 16
_NW = _NC * _NS

_CHUNK = 128

_M_BLK = 512
_N_BLK = 2048


def _make_gather(num_tokens, vocab, d_model, dtype):
    rows_per_w = num_tokens // _NW
    chunk = min(_CHUNK, rows_per_w)
    mesh = plsc.VectorSubcoreMesh(core_axis_name="c", subcore_axis_name="s")

    @functools.partial(
        pl.kernel,
        out_type=jax.ShapeDtypeStruct((num_tokens, d_model), dtype),
        mesh=mesh,
        scratch_types=[
            pltpu.VMEM((chunk,), jnp.int32),
            pltpu.VMEM((chunk, d_model), dtype),
            pltpu.SemaphoreType.DMA,
        ],
    )
    def gather(tokens_hbm, table_hbm, out_hbm, idx_v, rows_v, sem):
        wid = lax.axis_index("s") * _NC + lax.axis_index("c")
        base = wid * rows_per_w
        for c in range(rows_per_w // chunk):
            off = base + c * chunk
            pltpu.sync_copy(tokens_hbm.at[pl.ds(off, chunk)], idx_v)
            pltpu.async_copy(table_hbm.at[idx_v], rows_v, sem).wait()
            pltpu.sync_copy(rows_v, out_hbm.at[pl.ds(off, chunk)])

    return gather


def _matmul_body(h_ref, w_ref, b_ref, o_ref):
    h = h_ref[...].astype(jnp.bfloat16)
    w = w_ref[...].astype(jnp.bfloat16)
    acc = jnp.dot(h, w, preferred_element_type=jnp.float32)
    o_ref[...] = acc + b_ref[...]


def _make_matmul(num_tokens, d_model, vocab, dtype):
    m_tiles = num_tokens // _M_BLK
    n_tiles = vocab // _N_BLK
    return pl.pallas_call(
        _matmul_body,
        grid=(n_tiles, m_tiles),
        in_specs=[
            pl.BlockSpec((_M_BLK, d_model), lambda j, i: (i, 0)),
            pl.BlockSpec((d_model, _N_BLK), lambda j, i: (0, j)),
            pl.BlockSpec((1, _N_BLK), lambda j, i: (0, j)),
        ],
        out_specs=pl.BlockSpec((_M_BLK, _N_BLK), lambda j, i: (i, j)),
        out_shape=jax.ShapeDtypeStruct((num_tokens, vocab), dtype),
        compiler_params=pltpu.CompilerParams(
            dimension_semantics=("arbitrary", "arbitrary"),
        ),
    )


def kernel(tokens, emb_table, W, b):
    bsz, t = tokens.shape
    vocab, d_model = emb_table.shape
    num_tokens = bsz * t

    tok_flat = tokens.reshape(num_tokens).astype(jnp.int32)
    gather = _make_gather(num_tokens, vocab, d_model, emb_table.dtype)
    hidden = gather(tok_flat, emb_table)

    matmul = _make_matmul(num_tokens, d_model, vocab, W.dtype)
    logits = matmul(hidden, W, b.reshape(1, vocab))
    return logits.reshape(bsz, t, vocab)


# speedup vs baseline: 1.1691x; 1.1691x over previous
import functools

import jax
import jax.numpy as jnp
from jax import lax
from jax.experimental import pallas as pl
from jax.experimental.pallas import tpu as pltpu
from jax.experimental.pallas import tpu_sc as plsc

_NC = 2
_NS = 16
_NW = _NC * _NS

_CHUNK = 128

_M_BLK = 256


def _make_gather(num_tokens, vocab, d_model, dtype):
    rows_per_w = num_tokens // _NW
    chunk = min(_CHUNK, rows_per_w)
    mesh = plsc.VectorSubcoreMesh(core_axis_name="c", subcore_axis_name="s")

    @functools.partial(
        pl.kernel,
        out_type=jax.ShapeDtypeStruct((num_tokens, d_model), dtype),
        mesh=mesh,
        scratch_types=[
            pltpu.VMEM((chunk,), jnp.int32),
            pltpu.VMEM((chunk, d_model), dtype),
            pltpu.SemaphoreType.DMA,
        ],
    )
    def gather(tokens_hbm, table_hbm, out_hbm, idx_v, rows_v, sem):
        wid = lax.axis_index("s") * _NC + lax.axis_index("c")
        base = wid * rows_per_w
        for c in range(rows_per_w // chunk):
            off = base + c * chunk
            pltpu.sync_copy(tokens_hbm.at[pl.ds(off, chunk)], idx_v)
            pltpu.async_copy(table_hbm.at[idx_v], rows_v, sem).wait()
            pltpu.sync_copy(rows_v, out_hbm.at[pl.ds(off, chunk)])

    return gather


def _matmul_body(h_ref, w_ref, b_ref, o_ref):
    h = h_ref[...].astype(jnp.bfloat16)
    acc = jnp.dot(h, w_ref[...], preferred_element_type=jnp.float32)
    o_ref[...] = acc + b_ref[...]


def _make_matmul(num_tokens, d_model, vocab, dtype):
    m_tiles = num_tokens // _M_BLK
    return pl.pallas_call(
        _matmul_body,
        grid=(m_tiles,),
        in_specs=[
            pl.BlockSpec((_M_BLK, d_model), lambda i: (i, 0)),
            pl.BlockSpec((d_model, vocab), lambda i: (0, 0)),
            pl.BlockSpec((1, vocab), lambda i: (0, 0)),
        ],
        out_specs=pl.BlockSpec((_M_BLK, vocab), lambda i: (i, 0)),
        out_shape=jax.ShapeDtypeStruct((num_tokens, vocab), dtype),
        compiler_params=pltpu.CompilerParams(
            dimension_semantics=("arbitrary",),
        ),
    )


def kernel(tokens, emb_table, W, b):
    bsz, t = tokens.shape
    vocab, d_model = emb_table.shape
    num_tokens = bsz * t

    tok_flat = tokens.reshape(num_tokens).astype(jnp.int32)
    gather = _make_gather(num_tokens, vocab, d_model, emb_table.dtype)
    hidden = gather(tok_flat, emb_table)

    matmul = _make_matmul(num_tokens, d_model, vocab, W.dtype)
    logits = matmul(hidden, W.astype(jnp.bfloat16), b.reshape(1, vocab))
    return logits.reshape(bsz, t, vocab)


# speedup vs baseline: 1.1760x; 1.0059x over previous
import functools

import jax
import jax.numpy as jnp
from jax import lax
from jax.experimental import pallas as pl
from jax.experimental.pallas import tpu as pltpu
from jax.experimental.pallas import tpu_sc as plsc

_NC = 2
_NS = 16
_NW = _NC * _NS

_CHUNK = 64

_M_BLK = 256


def _make_gather(num_tokens, vocab, d_model, dtype):
    rows_per_w = num_tokens // _NW
    chunk = min(_CHUNK, rows_per_w)
    mesh = plsc.VectorSubcoreMesh(core_axis_name="c", subcore_axis_name="s")

    @functools.partial(
        pl.kernel,
        out_type=jax.ShapeDtypeStruct((num_tokens, d_model), dtype),
        mesh=mesh,
        scratch_types=[
            pltpu.VMEM((2, chunk), jnp.int32),
            pltpu.VMEM((2, chunk, d_model), dtype),
            pltpu.SemaphoreType.DMA,
            pltpu.SemaphoreType.DMA,
        ],
    )
    def gather(tokens_hbm, table_hbm, out_hbm, idx_v, rows_v, sem0, sem1):
        wid = lax.axis_index("s") * _NC + lax.axis_index("c")
        base = wid * rows_per_w
        n_chunks = rows_per_w // chunk
        sems = (sem0, sem1)

        def fire(c):
            s = c % 2
            off = base + c * chunk
            pltpu.sync_copy(tokens_hbm.at[pl.ds(off, chunk)], idx_v.at[s])
            return pltpu.async_copy(table_hbm.at[idx_v.at[s]], rows_v.at[s],
                                    sems[s])

        d = fire(0)
        for c in range(n_chunks):
            d.wait()
            if c + 1 < n_chunks:
                d = fire(c + 1)
            pltpu.sync_copy(rows_v.at[c % 2],
                            out_hbm.at[pl.ds(base + c * chunk, chunk)])

    return gather


def _matmul_body(h_ref, w_ref, b_ref, o_ref):
    h = h_ref[...].astype(jnp.bfloat16)
    acc = jnp.dot(h, w_ref[...], preferred_element_type=jnp.float32)
    o_ref[...] = acc + b_ref[...]


def _make_matmul(num_tokens, d_model, vocab, dtype):
    m_tiles = num_tokens // _M_BLK
    return pl.pallas_call(
        _matmul_body,
        grid=(m_tiles,),
        in_specs=[
            pl.BlockSpec((_M_BLK, d_model), lambda i: (i, 0)),
            pl.BlockSpec((d_model, vocab), lambda i: (0, 0)),
            pl.BlockSpec((1, vocab), lambda i: (0, 0)),
        ],
        out_specs=pl.BlockSpec((_M_BLK, vocab), lambda i: (i, 0)),
        out_shape=jax.ShapeDtypeStruct((num_tokens, vocab), dtype),
        compiler_params=pltpu.CompilerParams(
            dimension_semantics=("arbitrary",),
        ),
    )


def kernel(tokens, emb_table, W, b):
    bsz, t = tokens.shape
    vocab, d_model = emb_table.shape
    num_tokens = bsz * t

    tok_flat = tokens.reshape(num_tokens).astype(jnp.int32)
    gather = _make_gather(num_tokens, vocab, d_model, emb_table.dtype)
    hidden = gather(tok_flat, emb_table)

    matmul = _make_matmul(num_tokens, d_model, vocab, W.dtype)
    logits = matmul(hidden, W.astype(jnp.bfloat16), b.reshape(1, vocab))
    return logits.reshape(bsz, t, vocab)


# speedup vs baseline: 1.2038x; 1.0237x over previous
import functools

import jax
import jax.numpy as jnp
from jax import lax
from jax.experimental import pallas as pl
from jax.experimental.pallas import tpu as pltpu
from jax.experimental.pallas import tpu_sc as plsc

_NC = 2
_NS = 16
_NW = _NC * _NS

_CHUNK = 64

_M_BLK = 512


def _make_gather(num_tokens, vocab, d_model, dtype):
    rows_per_w = num_tokens // _NW
    chunk = min(_CHUNK, rows_per_w)
    mesh = plsc.VectorSubcoreMesh(core_axis_name="c", subcore_axis_name="s")

    @functools.partial(
        pl.kernel,
        out_type=jax.ShapeDtypeStruct((num_tokens, d_model), dtype),
        mesh=mesh,
        scratch_types=[
            pltpu.VMEM((2, chunk), jnp.int32),
            pltpu.VMEM((2, chunk, d_model), dtype),
            pltpu.SemaphoreType.DMA,
            pltpu.SemaphoreType.DMA,
        ],
    )
    def gather(tokens_hbm, table_hbm, out_hbm, idx_v, rows_v, sem0, sem1):
        wid = lax.axis_index("s") * _NC + lax.axis_index("c")
        base = wid * rows_per_w
        n_chunks = rows_per_w // chunk
        sems = (sem0, sem1)

        def fire(c):
            s = c % 2
            off = base + c * chunk
            pltpu.sync_copy(tokens_hbm.at[pl.ds(off, chunk)], idx_v.at[s])
            return pltpu.async_copy(table_hbm.at[idx_v.at[s]], rows_v.at[s],
                                    sems[s])

        d = fire(0)
        for c in range(n_chunks):
            d.wait()
            if c + 1 < n_chunks:
                d = fire(c + 1)
            pltpu.sync_copy(rows_v.at[c % 2],
                            out_hbm.at[pl.ds(base + c * chunk, chunk)])

    return gather


def _matmul_body(h_ref, w_ref, b_ref, o_ref):
    h = h_ref[...].astype(jnp.bfloat16)
    acc = jnp.dot(h, w_ref[...], preferred_element_type=jnp.float32)
    o_ref[...] = acc + b_ref[...]


def _make_matmul(num_tokens, d_model, vocab, dtype):
    m_tiles = num_tokens // _M_BLK
    return pl.pallas_call(
        _matmul_body,
        grid=(m_tiles,),
        in_specs=[
            pl.BlockSpec((_M_BLK, d_model), lambda i: (i, 0)),
            pl.BlockSpec((d_model, vocab), lambda i: (0, 0)),
            pl.BlockSpec((1, vocab), lambda i: (0, 0)),
        ],
        out_specs=pl.BlockSpec((_M_BLK, vocab), lambda i: (i, 0)),
        out_shape=jax.ShapeDtypeStruct((num_tokens, vocab), dtype),
        compiler_params=pltpu.CompilerParams(
            dimension_semantics=("arbitrary",),
        ),
    )


def kernel(tokens, emb_table, W, b):
    bsz, t = tokens.shape
    vocab, d_model = emb_table.shape
    num_tokens = bsz * t

    tok_flat = tokens.reshape(num_tokens).astype(jnp.int32)
    gather = _make_gather(num_tokens, vocab, d_model, emb_table.dtype)
    hidden = gather(tok_flat, emb_table)

    matmul = _make_matmul(num_tokens, d_model, vocab, W.dtype)
    logits = matmul(hidden, W.astype(jnp.bfloat16), b.reshape(1, vocab))
    return logits.reshape(bsz, t, vocab)
